# baseline (speedup 1.0000x reference)
import functools

import jax
import jax.numpy as jnp
from jax import lax
from jax.experimental import pallas as pl
from jax.experimental.pallas import tpu as pltpu
from jax.experimental.pallas import tpu_sc as plsc

_DO_SCALE = True

N = 10000
E = 320000
F_IN = 128
LATENT = 128
CLASSES = 16

NC = 2
NS = 16
E_CORE = E // NC
E_SUB = E_CORE // NS
CH = 80
NCHUNK = E_SUB // CH
ROWS_SUB = 624
ROWS_TAIL = N - NS * ROWS_SUB


@functools.lru_cache(maxsize=None)
def _make_sc_pass(feat):
    nslice = feat // 16

    mesh = plsc.VectorSubcoreMesh(core_axis_name="c", subcore_axis_name="s",
                                  num_cores=NC, num_subcores=NS)

    @functools.partial(
        pl.kernel,
        mesh=mesh,
        compiler_params=pltpu.CompilerParams(use_tc_tiling_on_sc=False),
        out_type=jax.ShapeDtypeStruct((NC * N, feat), jnp.float32),
        scratch_types=[
            pltpu.VMEM((NCHUNK, CH), jnp.int32),
            pltpu.VMEM((3, 2, CH), jnp.int32),
            pltpu.VMEM((CH, feat), jnp.float32),
            pltpu.VMEM((CH, feat), jnp.float32),
            pltpu.VMEM((CH, feat), jnp.float32),
            pltpu.VMEM_SHARED((N, feat), jnp.float32),
            pltpu.SemaphoreType.DMA,
            pltpu.SemaphoreType.DMA,
            pltpu.SemaphoreType.DMA,
            pltpu.SemaphoreType.DMA,
            pltpu.SemaphoreType.DMA,
            pltpu.SemaphoreType.DMA,
            pltpu.SemaphoreType.DMA,
            pltpu.SemaphoreType.DMA,
            pltpu.SemaphoreType.DMA,
        ],
    )
    def sc_pass(x_hbm, src_hbm, de_hbm, out_hbm,
                src_all, deb, rows0_v, rows1_v, rows2_v, acc_sh,
                gsem0, gsem1, gsem2, ssem0, ssem1, ssem2,
                esem0, esem1, esem2):
        c = lax.axis_index("c")
        s = lax.axis_index("s")
        rows = (rows0_v, rows1_v, rows2_v)
        gsems = (gsem0, gsem1, gsem2)
        ssems = (ssem0, ssem1, ssem2)
        esems = (esem0, esem1, esem2)
        rows_v = rows0_v

        pltpu.sync_copy(src_hbm.at[c, s], src_all)

        def zrow(j, carry):
            for k in range(nslice):
                rows_v[j, pl.ds(k * 16, 16)] = jnp.zeros((16,), jnp.float32)
            return carry
        lax.fori_loop(0, CH, zrow, 0)
        stripe = s * ROWS_SUB
        nfull = ROWS_SUB // CH
        for i in range(nfull):
            pltpu.sync_copy(rows_v, acc_sh.at[pl.ds(stripe + i * CH, CH)])
        rem = ROWS_SUB - nfull * CH
        if rem:
            pltpu.sync_copy(rows_v.at[pl.ds(0, rem)],
                            acc_sh.at[pl.ds(stripe + nfull * CH, rem)])

        @pl.when(s == NS - 1)
        def _zero_tail():
            pltpu.sync_copy(rows_v.at[pl.ds(0, ROWS_TAIL)],
                            acc_sh.at[pl.ds(NS * ROWS_SUB, ROWS_TAIL)])
        plsc.subcore_barrier()

        gbytes = CH * feat * 4
        ebytes = 2 * CH * 4

        def g_issue(i, b):
            pltpu.async_copy(x_hbm.at[src_all.at[i]], rows[b], gsems[b])

        def e_issue(i, b):
            pltpu.async_copy(de_hbm.at[c, s, i], deb.at[b], esems[b])

        def s_issue(b):
            pltpu.async_copy(rows[b], acc_sh.at[deb.at[b, 0]], ssems[b],
                             add=True)

        def s_wait(b):
            pltpu.make_async_copy(x_hbm.at[pl.ds(0, CH)], rows[b],
                                  ssems[b]).wait()

        def g_wait(b):
            pltpu.make_async_copy(x_hbm.at[pl.ds(0, CH)], rows[b],
                                  gsems[b]).wait()

        def e_wait(b):
            pltpu.make_async_copy(de_hbm.at[0, 0, 0], deb.at[b],
                                  esems[b]).wait()

        def process(i, b, refill=True, prev=True):
            g_wait(b)
            e_wait(b)

            def scale(g, carry2):
                wv = jax.lax.bitcast_convert_type(
                    deb[b, 1, pl.ds(g * 16, 16)], jnp.float32)
                for j2 in range(16):
                    w = wv[j2]
                    row = g * 16 + j2
                    for k in range(nslice):
                        sl = pl.ds(k * 16, 16)
                        rows[b][row, sl] = rows[b][row, sl] * w
                return carry2
            if _DO_SCALE:
                lax.fori_loop(0, CH // 16, scale, 0)

            s_issue(b)
            if refill:
                if prev:
                    s_wait((b + 2) % 3)
                g_issue(i + 2, (b + 2) % 3)
                e_issue(i + 2, (b + 2) % 3)

        g_issue(0, 0)
        e_issue(0, 0)
        g_issue(1, 1)
        e_issue(1, 1)
        process(0, 0, prev=False)
        process(1, 1)
        process(2, 2)

        def triple(o, carry):
            i = 3 * o
            process(i, 0)
            process(i + 1, 1)
            process(i + 2, 2)
            return carry
        lax.fori_loop(1, NCHUNK // 3, triple, 0)
        for i in range(3 * (NCHUNK // 3), NCHUNK):
            process(i, i % 3, refill=False)
        s_wait((NCHUNK - 3) % 3)
        s_wait((NCHUNK - 2) % 3)
        s_wait((NCHUNK - 1) % 3)

        plsc.subcore_barrier()
        pltpu.sync_copy(acc_sh.at[pl.ds(stripe, ROWS_SUB)],
                        out_hbm.at[pl.ds(c * N + stripe, ROWS_SUB)])

        @pl.when(s == NS - 1)
        def _write_tail():
            pltpu.sync_copy(acc_sh.at[pl.ds(NS * ROWS_SUB, ROWS_TAIL)],
                            out_hbm.at[pl.ds(c * N + NS * ROWS_SUB, ROWS_TAIL)])

    return sc_pass


_BLK = 2000


def _mid_body(p0_ref, p1_ref, w1_ref, b1_ref, w2_ref,
              out1_ref, ixz1_ref, h2_ref):
    s = p0_ref[...] + p1_ref[...]
    out1 = jax.lax.dot_general(
        s, w1_ref[...], (((1,), (0,)), ((), ())),
        preferred_element_type=jnp.float32) + b1_ref[...]
    out1_ref[...] = out1
    half = LATENT // 2
    mean = out1[:, :half]
    sa = out1[:, half:]
    std = jnp.maximum(sa, 0.0) + jnp.log1p(jnp.exp(-jnp.abs(sa))) + 1e-10
    ixz1_ref[...] = -jnp.log(std) + (std * std + mean * mean) * 0.5 - 0.5
    h2_ref[...] = jax.lax.dot_general(
        out1, w2_ref[...], (((1,), (0,)), ((), ())),
        preferred_element_type=jnp.float32)


def _tc_mid(p0, p1, W1, b1, W2):
    grid = (N // _BLK,)
    return pl.pallas_call(
        _mid_body,
        grid=grid,
        in_specs=[
            pl.BlockSpec((_BLK, LATENT), lambda i: (i, 0)),
            pl.BlockSpec((_BLK, LATENT), lambda i: (i + N // _BLK, 0)),
            pl.BlockSpec((F_IN, LATENT), lambda i: (0, 0)),
            pl.BlockSpec((1, LATENT), lambda i: (0, 0)),
            pl.BlockSpec((LATENT, CLASSES), lambda i: (0, 0)),
        ],
        out_specs=[
            pl.BlockSpec((_BLK, LATENT), lambda i: (i, 0)),
            pl.BlockSpec((_BLK, LATENT // 2), lambda i: (i, 0)),
            pl.BlockSpec((_BLK, CLASSES), lambda i: (i, 0)),
        ],
        out_shape=[
            jax.ShapeDtypeStruct((N, LATENT), jnp.float32),
            jax.ShapeDtypeStruct((N, LATENT // 2), jnp.float32),
            jax.ShapeDtypeStruct((N, CLASSES), jnp.float32),
        ],
    )(p0, p1, W1, b1.reshape(1, LATENT), W2)


def _final_body(q0_ref, q1_ref, b2_ref, out2_ref, ixz2_ref):
    out2 = q0_ref[...] + q1_ref[...] + b2_ref[...]
    out2_ref[...] = out2
    half = CLASSES // 2
    mean = out2[:, :half]
    sa = out2[:, half:]
    std = jnp.maximum(sa, 0.0) + jnp.log1p(jnp.exp(-jnp.abs(sa))) + 1e-10
    ixz2_ref[...] = -jnp.log(std) + (std * std + mean * mean) * 0.5 - 0.5


def _tc_final(q0, q1, b2):
    grid = (N // _BLK,)
    return pl.pallas_call(
        _final_body,
        grid=grid,
        in_specs=[
            pl.BlockSpec((_BLK, CLASSES), lambda i: (i, 0)),
            pl.BlockSpec((_BLK, CLASSES), lambda i: (i + N // _BLK, 0)),
            pl.BlockSpec((1, CLASSES), lambda i: (0, 0)),
        ],
        out_specs=[
            pl.BlockSpec((_BLK, CLASSES), lambda i: (i, 0)),
            pl.BlockSpec((_BLK, CLASSES // 2), lambda i: (i, 0)),
        ],
        out_shape=[
            jax.ShapeDtypeStruct((N, CLASSES), jnp.float32),
            jax.ShapeDtypeStruct((N, CLASSES // 2), jnp.float32),
        ],
    )(q0, q1, b2.reshape(1, CLASSES))


def kernel(x, edge_index, edge_attr, W1, b1, W2, b2):
    eshape = (NC, NS, NCHUNK, CH)
    src = edge_index[0].reshape(eshape)
    de = jnp.stack(
        [edge_index[1].reshape(eshape),
         jax.lax.bitcast_convert_type(edge_attr, jnp.int32).reshape(eshape)],
        axis=3)

    p = _make_sc_pass(F_IN)(x, src, de)
    out1, ixz1, h2 = _tc_mid(p, p, W1, b1, W2)
    q = _make_sc_pass(CLASSES)(h2, src, de)
    out2, ixz2 = _tc_final(q, q, b2)

    skl1 = jnp.zeros_like(ixz1)
    skl2 = jnp.zeros_like(ixz2)
    return (out2, out1, ixz1, skl1, ixz2, skl2)


# speedup vs baseline: 1.1079x; 1.1079x over previous
import functools

import jax
import jax.numpy as jnp
from jax import lax
from jax.experimental import pallas as pl
from jax.experimental.pallas import tpu as pltpu
from jax.experimental.pallas import tpu_sc as plsc

_DO_SCALE = True

N = 10000
E = 320000
F_IN = 128
LATENT = 128
CLASSES = 16

NC = 2
NS = 16
E_CORE = E // NC
E_SUB = E_CORE // NS
CH = 80
NCHUNK = E_SUB // CH
ROWS_SUB = 624
ROWS_TAIL = N - NS * ROWS_SUB


@functools.lru_cache(maxsize=None)
def _make_sc_pass(feat):
    nslice = feat // 16

    mesh = plsc.VectorSubcoreMesh(core_axis_name="c", subcore_axis_name="s",
                                  num_cores=NC, num_subcores=NS)

    @functools.partial(
        pl.kernel,
        mesh=mesh,
        compiler_params=pltpu.CompilerParams(use_tc_tiling_on_sc=False),
        out_type=jax.ShapeDtypeStruct((NC * N, feat), jnp.float32),
        scratch_types=[
            pltpu.VMEM((NCHUNK, CH), jnp.int32),
            pltpu.VMEM((3, CH), jnp.int32),
            pltpu.VMEM((3, CH), jnp.float32),
            pltpu.VMEM((CH, feat), jnp.float32),
            pltpu.VMEM((CH, feat), jnp.float32),
            pltpu.VMEM((CH, feat), jnp.float32),
            pltpu.VMEM_SHARED((N, feat), jnp.float32),
            pltpu.SemaphoreType.DMA,
            pltpu.SemaphoreType.DMA,
            pltpu.SemaphoreType.DMA,
            pltpu.SemaphoreType.DMA,
            pltpu.SemaphoreType.DMA,
            pltpu.SemaphoreType.DMA,
            pltpu.SemaphoreType.DMA,
            pltpu.SemaphoreType.DMA,
            pltpu.SemaphoreType.DMA,
        ],
    )
    def sc_pass(x_hbm, src_hbm, dst_hbm, ew_hbm, out_hbm,
                src_all, dstb, ewb, rows0_v, rows1_v, rows2_v, acc_sh,
                gsem0, gsem1, gsem2, ssem0, ssem1, ssem2,
                esem0, esem1, esem2):
        c = lax.axis_index("c")
        s = lax.axis_index("s")
        rows = (rows0_v, rows1_v, rows2_v)
        gsems = (gsem0, gsem1, gsem2)
        ssems = (ssem0, ssem1, ssem2)
        esems = (esem0, esem1, esem2)
        rows_v = rows0_v

        pltpu.sync_copy(src_hbm.at[c, s], src_all)

        def zrow(j, carry):
            for k in range(nslice):
                rows_v[j, pl.ds(k * 16, 16)] = jnp.zeros((16,), jnp.float32)
            return carry
        lax.fori_loop(0, CH, zrow, 0)
        stripe = s * ROWS_SUB
        nfull = ROWS_SUB // CH
        for i in range(nfull):
            pltpu.sync_copy(rows_v, acc_sh.at[pl.ds(stripe + i * CH, CH)])
        rem = ROWS_SUB - nfull * CH
        if rem:
            pltpu.sync_copy(rows_v.at[pl.ds(0, rem)],
                            acc_sh.at[pl.ds(stripe + nfull * CH, rem)])

        @pl.when(s == NS - 1)
        def _zero_tail():
            pltpu.sync_copy(rows_v.at[pl.ds(0, ROWS_TAIL)],
                            acc_sh.at[pl.ds(NS * ROWS_SUB, ROWS_TAIL)])
        plsc.subcore_barrier()

        gbytes = CH * feat * 4
        ebytes = 2 * CH * 4

        def g_issue(i, b):
            pltpu.async_copy(x_hbm.at[src_all.at[i]], rows[b], gsems[b])

        def e_issue(i, b):
            pltpu.async_copy(dst_hbm.at[c, s, i], dstb.at[b], esems[b])
            pltpu.async_copy(ew_hbm.at[c, s, i], ewb.at[b], esems[b])

        def s_issue(b):
            pltpu.async_copy(rows[b], acc_sh.at[dstb.at[b]], ssems[b],
                             add=True)

        def s_wait(b):
            pltpu.make_async_copy(x_hbm.at[pl.ds(0, CH)], rows[b],
                                  ssems[b]).wait()

        def g_wait(b):
            pltpu.make_async_copy(x_hbm.at[pl.ds(0, CH)], rows[b],
                                  gsems[b]).wait()

        def e_wait(b):
            pltpu.make_async_copy(dst_hbm.at[0, 0, 0], dstb.at[b],
                                  esems[b]).wait()
            pltpu.make_async_copy(dst_hbm.at[0, 0, 0], ewb.at[b],
                                  esems[b]).wait()

        def process(i, b, refill=True, prev=True):
            g_wait(b)
            e_wait(b)

            def scale(g, carry2):
                wv = ewb[b, pl.ds(g * 16, 16)]
                for j2 in range(16):
                    w = wv[j2]
                    row = g * 16 + j2
                    for k in range(nslice):
                        sl = pl.ds(k * 16, 16)
                        rows[b][row, sl] = rows[b][row, sl] * w
                return carry2
            if _DO_SCALE:
                lax.fori_loop(0, CH // 16, scale, 0)

            s_issue(b)
            if refill:
                if prev:
                    s_wait((b + 2) % 3)
                g_issue(i + 2, (b + 2) % 3)
                e_issue(i + 2, (b + 2) % 3)

        g_issue(0, 0)
        e_issue(0, 0)
        g_issue(1, 1)
        e_issue(1, 1)
        process(0, 0, prev=False)
        process(1, 1)
        process(2, 2)

        def triple(o, carry):
            i = 3 * o
            process(i, 0)
            process(i + 1, 1)
            process(i + 2, 2)
            return carry
        lax.fori_loop(1, NCHUNK // 3, triple, 0)
        for i in range(3 * (NCHUNK // 3), NCHUNK):
            process(i, i % 3, refill=False)
        s_wait((NCHUNK - 3) % 3)
        s_wait((NCHUNK - 2) % 3)
        s_wait((NCHUNK - 1) % 3)

        plsc.subcore_barrier()
        pltpu.sync_copy(acc_sh.at[pl.ds(stripe, ROWS_SUB)],
                        out_hbm.at[pl.ds(c * N + stripe, ROWS_SUB)])

        @pl.when(s == NS - 1)
        def _write_tail():
            pltpu.sync_copy(acc_sh.at[pl.ds(NS * ROWS_SUB, ROWS_TAIL)],
                            out_hbm.at[pl.ds(c * N + NS * ROWS_SUB, ROWS_TAIL)])

    return sc_pass


_BLK = 2000


def _mid_body(p0_ref, p1_ref, w1_ref, b1_ref, w2_ref,
              out1_ref, ixz1_ref, h2_ref):
    s = p0_ref[...] + p1_ref[...]
    out1 = jax.lax.dot_general(
        s, w1_ref[...], (((1,), (0,)), ((), ())),
        preferred_element_type=jnp.float32) + b1_ref[...]
    out1_ref[...] = out1
    half = LATENT // 2
    mean = out1[:, :half]
    sa = out1[:, half:]
    std = jnp.maximum(sa, 0.0) + jnp.log1p(jnp.exp(-jnp.abs(sa))) + 1e-10
    ixz1_ref[...] = -jnp.log(std) + (std * std + mean * mean) * 0.5 - 0.5
    h2_ref[...] = jax.lax.dot_general(
        out1, w2_ref[...], (((1,), (0,)), ((), ())),
        preferred_element_type=jnp.float32)


def _tc_mid(p0, p1, W1, b1, W2):
    grid = (N // _BLK,)
    return pl.pallas_call(
        _mid_body,
        grid=grid,
        in_specs=[
            pl.BlockSpec((_BLK, LATENT), lambda i: (i, 0)),
            pl.BlockSpec((_BLK, LATENT), lambda i: (i + N // _BLK, 0)),
            pl.BlockSpec((F_IN, LATENT), lambda i: (0, 0)),
            pl.BlockSpec((1, LATENT), lambda i: (0, 0)),
            pl.BlockSpec((LATENT, CLASSES), lambda i: (0, 0)),
        ],
        out_specs=[
            pl.BlockSpec((_BLK, LATENT), lambda i: (i, 0)),
            pl.BlockSpec((_BLK, LATENT // 2), lambda i: (i, 0)),
            pl.BlockSpec((_BLK, CLASSES), lambda i: (i, 0)),
        ],
        out_shape=[
            jax.ShapeDtypeStruct((N, LATENT), jnp.float32),
            jax.ShapeDtypeStruct((N, LATENT // 2), jnp.float32),
            jax.ShapeDtypeStruct((N, CLASSES), jnp.float32),
        ],
    )(p0, p1, W1, b1.reshape(1, LATENT), W2)


def _final_body(q0_ref, q1_ref, b2_ref, out2_ref, ixz2_ref):
    out2 = q0_ref[...] + q1_ref[...] + b2_ref[...]
    out2_ref[...] = out2
    half = CLASSES // 2
    mean = out2[:, :half]
    sa = out2[:, half:]
    std = jnp.maximum(sa, 0.0) + jnp.log1p(jnp.exp(-jnp.abs(sa))) + 1e-10
    ixz2_ref[...] = -jnp.log(std) + (std * std + mean * mean) * 0.5 - 0.5


def _tc_final(q0, q1, b2):
    grid = (N // _BLK,)
    return pl.pallas_call(
        _final_body,
        grid=grid,
        in_specs=[
            pl.BlockSpec((_BLK, CLASSES), lambda i: (i, 0)),
            pl.BlockSpec((_BLK, CLASSES), lambda i: (i + N // _BLK, 0)),
            pl.BlockSpec((1, CLASSES), lambda i: (0, 0)),
        ],
        out_specs=[
            pl.BlockSpec((_BLK, CLASSES), lambda i: (i, 0)),
            pl.BlockSpec((_BLK, CLASSES // 2), lambda i: (i, 0)),
        ],
        out_shape=[
            jax.ShapeDtypeStruct((N, CLASSES), jnp.float32),
            jax.ShapeDtypeStruct((N, CLASSES // 2), jnp.float32),
        ],
    )(q0, q1, b2.reshape(1, CLASSES))


def kernel(x, edge_index, edge_attr, W1, b1, W2, b2):
    eshape = (NC, NS, NCHUNK, CH)
    src = edge_index[0].reshape(eshape)
    dst = edge_index[1].reshape(eshape)
    ew = edge_attr.reshape(eshape)

    p = _make_sc_pass(F_IN)(x, src, dst, ew)
    out1, ixz1, h2 = _tc_mid(p, p, W1, b1, W2)
    q = _make_sc_pass(CLASSES)(h2, src, dst, ew)
    out2, ixz2 = _tc_final(q, q, b2)

    skl1 = jnp.zeros_like(ixz1)
    skl2 = jnp.zeros_like(ixz2)
    return (out2, out1, ixz1, skl1, ixz2, skl2)


# speedup vs baseline: 1.1576x; 1.0449x over previous
import functools

import jax
import jax.numpy as jnp
from jax import lax
from jax.experimental import pallas as pl
from jax.experimental.pallas import tpu as pltpu
from jax.experimental.pallas import tpu_sc as plsc

N = 10000
E = 320000
F_IN = 128
LATENT = 128
CLASSES = 16

NC = 2
NS = 16
E_CORE = E // NC
E_SUB = E_CORE // NS
CH_WIDE = 80
CH_CLS = 125
ROWS_SUB = 624
ROWS_TAIL = N - NS * ROWS_SUB


@functools.lru_cache(maxsize=None)
def _make_sc_pass(feat, CH):
    nslice = feat // 16
    NCHUNK = E_SUB // CH

    mesh = plsc.VectorSubcoreMesh(core_axis_name="c", subcore_axis_name="s",
                                  num_cores=NC, num_subcores=NS)

    @functools.partial(
        pl.kernel,
        mesh=mesh,
        compiler_params=pltpu.CompilerParams(use_tc_tiling_on_sc=False),
        out_type=jax.ShapeDtypeStruct((NC * N, feat), jnp.float32),
        scratch_types=[
            pltpu.VMEM((NCHUNK, CH), jnp.int32),
            pltpu.VMEM((3, CH), jnp.int32),
            pltpu.VMEM((3, CH), jnp.float32),
            pltpu.VMEM((CH, feat), jnp.float32),
            pltpu.VMEM((CH, feat), jnp.float32),
            pltpu.VMEM((CH, feat), jnp.float32),
            pltpu.VMEM_SHARED((N, feat), jnp.float32),
            pltpu.SemaphoreType.DMA,
            pltpu.SemaphoreType.DMA,
            pltpu.SemaphoreType.DMA,
            pltpu.SemaphoreType.DMA,
            pltpu.SemaphoreType.DMA,
            pltpu.SemaphoreType.DMA,
            pltpu.SemaphoreType.DMA,
            pltpu.SemaphoreType.DMA,
            pltpu.SemaphoreType.DMA,
        ],
    )
    def sc_pass(x_hbm, src_hbm, dst_hbm, ew_hbm, out_hbm,
                src_all, dstb, ewb, rows0_v, rows1_v, rows2_v, acc_sh,
                gsem0, gsem1, gsem2, ssem0, ssem1, ssem2,
                esem0, esem1, esem2):
        c = lax.axis_index("c")
        s = lax.axis_index("s")
        rows = (rows0_v, rows1_v, rows2_v)
        gsems = (gsem0, gsem1, gsem2)
        ssems = (ssem0, ssem1, ssem2)
        esems = (esem0, esem1, esem2)
        rows_v = rows0_v

        pltpu.sync_copy(src_hbm.at[c, s], src_all)

        def zrow(j, carry):
            for k in range(nslice):
                rows_v[j, pl.ds(k * 16, 16)] = jnp.zeros((16,), jnp.float32)
            return carry
        lax.fori_loop(0, CH, zrow, 0)
        stripe = s * ROWS_SUB
        nfull = ROWS_SUB // CH
        for i in range(nfull):
            pltpu.sync_copy(rows_v, acc_sh.at[pl.ds(stripe + i * CH, CH)])
        rem = ROWS_SUB - nfull * CH
        if rem:
            pltpu.sync_copy(rows_v.at[pl.ds(0, rem)],
                            acc_sh.at[pl.ds(stripe + nfull * CH, rem)])

        @pl.when(s == NS - 1)
        def _zero_tail():
            pltpu.sync_copy(rows_v.at[pl.ds(0, ROWS_TAIL)],
                            acc_sh.at[pl.ds(NS * ROWS_SUB, ROWS_TAIL)])
        plsc.subcore_barrier()

        gbytes = CH * feat * 4
        ebytes = 2 * CH * 4

        def g_issue(i, b):
            pltpu.async_copy(x_hbm.at[src_all.at[i]], rows[b], gsems[b])

        def e_issue(i, b):
            pltpu.async_copy(dst_hbm.at[c, s, i], dstb.at[b], esems[b])
            pltpu.async_copy(ew_hbm.at[c, s, i], ewb.at[b], esems[b])

        def s_issue(b):
            pltpu.async_copy(rows[b], acc_sh.at[dstb.at[b]], ssems[b],
                             add=True)

        def s_wait(b):
            pltpu.make_async_copy(x_hbm.at[pl.ds(0, CH)], rows[b],
                                  ssems[b]).wait()

        def g_wait(b):
            pltpu.make_async_copy(x_hbm.at[pl.ds(0, CH)], rows[b],
                                  gsems[b]).wait()

        def e_wait(b):
            pltpu.make_async_copy(dst_hbm.at[0, 0, 0], dstb.at[b],
                                  esems[b]).wait()
            pltpu.make_async_copy(dst_hbm.at[0, 0, 0], ewb.at[b],
                                  esems[b]).wait()

        def process(i, b, refill=True, prev=True):
            g_wait(b)
            e_wait(b)

            def scale(g, carry2):
                wv = ewb[b, pl.ds(g * 16, 16)]
                for j2 in range(16):
                    w = wv[j2]
                    row = g * 16 + j2
                    for k in range(nslice):
                        sl = pl.ds(k * 16, 16)
                        rows[b][row, sl] = rows[b][row, sl] * w
                return carry2
            lax.fori_loop(0, CH // 16, scale, 0)
            if CH % 16:
                wv2 = ewb[b, pl.ds(CH - 16, 16)]
                for j2 in range(16 - CH % 16, 16):
                    w = wv2[j2]
                    row = CH - 16 + j2
                    for k in range(nslice):
                        sl = pl.ds(k * 16, 16)
                        rows[b][row, sl] = rows[b][row, sl] * w

            s_issue(b)
            if refill:
                if prev:
                    s_wait((b + 2) % 3)
                g_issue(i + 2, (b + 2) % 3)
                e_issue(i + 2, (b + 2) % 3)

        g_issue(0, 0)
        e_issue(0, 0)
        g_issue(1, 1)
        e_issue(1, 1)
        process(0, 0, prev=False)
        process(1, 1)
        process(2, 2)

        def triple(o, carry):
            i = 3 * o
            process(i, 0)
            process(i + 1, 1)
            process(i + 2, 2)
            return carry
        lax.fori_loop(1, NCHUNK // 3, triple, 0)
        for i in range(3 * (NCHUNK // 3), NCHUNK):
            process(i, i % 3, refill=False)
        s_wait((NCHUNK - 3) % 3)
        s_wait((NCHUNK - 2) % 3)
        s_wait((NCHUNK - 1) % 3)

        plsc.subcore_barrier()
        pltpu.sync_copy(acc_sh.at[pl.ds(stripe, ROWS_SUB)],
                        out_hbm.at[pl.ds(c * N + stripe, ROWS_SUB)])

        @pl.when(s == NS - 1)
        def _write_tail():
            pltpu.sync_copy(acc_sh.at[pl.ds(NS * ROWS_SUB, ROWS_TAIL)],
                            out_hbm.at[pl.ds(c * N + NS * ROWS_SUB, ROWS_TAIL)])

    return sc_pass


_BLK = 2000


def _mid_body(p0_ref, p1_ref, w1_ref, b1_ref, w2_ref,
              out1_ref, ixz1_ref, h2_ref):
    s = p0_ref[...] + p1_ref[...]
    out1 = jax.lax.dot_general(
        s, w1_ref[...], (((1,), (0,)), ((), ())),
        preferred_element_type=jnp.float32) + b1_ref[...]
    out1_ref[...] = out1
    half = LATENT // 2
    mean = out1[:, :half]
    sa = out1[:, half:]
    std = jnp.maximum(sa, 0.0) + jnp.log1p(jnp.exp(-jnp.abs(sa))) + 1e-10
    ixz1_ref[...] = -jnp.log(std) + (std * std + mean * mean) * 0.5 - 0.5
    h2_ref[...] = jax.lax.dot_general(
        out1, w2_ref[...], (((1,), (0,)), ((), ())),
        preferred_element_type=jnp.float32)


def _tc_mid(p0, p1, W1, b1, W2):
    grid = (N // _BLK,)
    return pl.pallas_call(
        _mid_body,
        grid=grid,
        in_specs=[
            pl.BlockSpec((_BLK, LATENT), lambda i: (i, 0)),
            pl.BlockSpec((_BLK, LATENT), lambda i: (i + N // _BLK, 0)),
            pl.BlockSpec((F_IN, LATENT), lambda i: (0, 0)),
            pl.BlockSpec((1, LATENT), lambda i: (0, 0)),
            pl.BlockSpec((LATENT, CLASSES), lambda i: (0, 0)),
        ],
        out_specs=[
            pl.BlockSpec((_BLK, LATENT), lambda i: (i, 0)),
            pl.BlockSpec((_BLK, LATENT // 2), lambda i: (i, 0)),
            pl.BlockSpec((_BLK, CLASSES), lambda i: (i, 0)),
        ],
        out_shape=[
            jax.ShapeDtypeStruct((N, LATENT), jnp.float32),
            jax.ShapeDtypeStruct((N, LATENT // 2), jnp.float32),
            jax.ShapeDtypeStruct((N, CLASSES), jnp.float32),
        ],
    )(p0, p1, W1, b1.reshape(1, LATENT), W2)


def _final_body(q0_ref, q1_ref, b2_ref, out2_ref, ixz2_ref):
    out2 = q0_ref[...] + q1_ref[...] + b2_ref[...]
    out2_ref[...] = out2
    half = CLASSES // 2
    mean = out2[:, :half]
    sa = out2[:, half:]
    std = jnp.maximum(sa, 0.0) + jnp.log1p(jnp.exp(-jnp.abs(sa))) + 1e-10
    ixz2_ref[...] = -jnp.log(std) + (std * std + mean * mean) * 0.5 - 0.5


def _tc_final(q0, q1, b2):
    grid = (N // _BLK,)
    return pl.pallas_call(
        _final_body,
        grid=grid,
        in_specs=[
            pl.BlockSpec((_BLK, CLASSES), lambda i: (i, 0)),
            pl.BlockSpec((_BLK, CLASSES), lambda i: (i + N // _BLK, 0)),
            pl.BlockSpec((1, CLASSES), lambda i: (0, 0)),
        ],
        out_specs=[
            pl.BlockSpec((_BLK, CLASSES), lambda i: (i, 0)),
            pl.BlockSpec((_BLK, CLASSES // 2), lambda i: (i, 0)),
        ],
        out_shape=[
            jax.ShapeDtypeStruct((N, CLASSES), jnp.float32),
            jax.ShapeDtypeStruct((N, CLASSES // 2), jnp.float32),
        ],
    )(q0, q1, b2.reshape(1, CLASSES))


def kernel(x, edge_index, edge_attr, W1, b1, W2, b2):
    esh1 = (NC, NS, E_SUB // CH_WIDE, CH_WIDE)
    esh2 = (NC, NS, E_SUB // CH_CLS, CH_CLS)
    src = edge_index[0]
    dst = edge_index[1]
    ew = edge_attr

    p = _make_sc_pass(F_IN, CH_WIDE)(
        x, src.reshape(esh1), dst.reshape(esh1), ew.reshape(esh1))
    out1, ixz1, h2 = _tc_mid(p, p, W1, b1, W2)
    q = _make_sc_pass(CLASSES, CH_CLS)(
        h2, src.reshape(esh2), dst.reshape(esh2), ew.reshape(esh2))
    out2, ixz2 = _tc_final(q, q, b2)

    skl1 = jnp.zeros_like(ixz1)
    skl2 = jnp.zeros_like(ixz2)
    return (out2, out1, ixz1, skl1, ixz2, skl2)
